# baseline (speedup 1.0000x reference)
import functools

import jax
import jax.numpy as jnp
from jax import lax
from jax.experimental import pallas as pl
from jax.experimental.pallas import tpu as pltpu
from jax.experimental.pallas import tpu_sc as plsc

_BATCH = 16384
_N_SLOTS = 26
_VOCAB = 100000
_EMB = 32

_ROWS = _BATCH * _N_SLOTS
_IW = 128
_NROW128 = _ROWS // _IW
_NW = 32
_PER_W = _NROW128 // _NW
_CHUNK = 8
_NCHUNK = _PER_W // _CHUNK


def _body(tab_hbm, idx_hbm, out_hbm, idx_v, rows_v, sem):
    nc = 2
    wid = lax.axis_index("s") * nc + lax.axis_index("c")
    wstart = wid * _PER_W

    def chunk(c):
        row0 = wstart + c * _CHUNK
        pltpu.sync_copy(idx_hbm.at[pl.ds(row0, _CHUNK)], idx_v)
        lane = lax.iota(jnp.int32, 16)
        for j in range(_CHUNK):
            fbase = (row0 + j) * _IW
            for k in range(_IW // 16):
                fpos = fbase + k * 16 + lane
                slot = lax.rem(fpos, _N_SLOTS)
                raw = idx_v[j, pl.ds(k * 16, 16)]
                idx_v[j, pl.ds(k * 16, 16)] = raw + slot * _VOCAB
        copies = [
            pltpu.async_copy(tab_hbm.at[idx_v.at[j]], rows_v.at[j], sem)
            for j in range(_CHUNK)
        ]
        for cp in copies:
            cp.wait()
        pltpu.sync_copy(rows_v, out_hbm.at[pl.ds(row0, _CHUNK)])

    pl.loop(0, _NCHUNK)(chunk)


def kernel(input, tables):
    tab = tables.reshape(_N_SLOTS * _VOCAB, _EMB)
    idx = input.reshape(_NROW128, _IW)
    mesh = plsc.VectorSubcoreMesh(core_axis_name="c", subcore_axis_name="s")
    k = functools.partial(
        pl.kernel,
        out_type=jax.ShapeDtypeStruct((_NROW128, _IW, _EMB), jnp.float32),
        mesh=mesh,
        scratch_types=[
            pltpu.VMEM((_CHUNK, _IW), jnp.int32),
            pltpu.VMEM((_CHUNK, _IW, _EMB), jnp.float32),
            pltpu.SemaphoreType.DMA,
        ],
        compiler_params=pltpu.CompilerParams(use_tc_tiling_on_sc=False),
    )(_body)
    out = k(tab, idx)
    return out.reshape(_BATCH, _N_SLOTS, _EMB)


# speedup vs baseline: 1.0058x; 1.0058x over previous
import functools

import jax
import jax.numpy as jnp
from jax import lax
from jax.experimental import pallas as pl
from jax.experimental.pallas import tpu as pltpu
from jax.experimental.pallas import tpu_sc as plsc

_BATCH = 16384
_N_SLOTS = 26
_VOCAB = 100000
_EMB = 32

_ROWS = _BATCH * _N_SLOTS
_IW = 128
_NROW128 = _ROWS // _IW
_NW = 32
_PER_W = _NROW128 // _NW
_CHUNK = 8
_NCHUNK = _PER_W // _CHUNK


def _body(tab_hbm, idx_hbm, out_hbm, idx_v, rows_v, gsem, wsem):
    nc = 2
    wid = lax.axis_index("s") * nc + lax.axis_index("c")
    wstart = wid * _PER_W

    def stage_indices(g, s):
        row0 = wstart + g * _CHUNK
        pltpu.sync_copy(idx_hbm.at[pl.ds(row0, _CHUNK)], idx_v.at[s])
        lane = lax.iota(jnp.int32, 16)
        for j in range(_CHUNK):
            fbase = (row0 + j) * _IW
            for k in range(_IW // 16):
                fpos = fbase + k * 16 + lane
                slot = lax.rem(fpos, _N_SLOTS)
                raw = idx_v[s, j, pl.ds(k * 16, 16)]
                idx_v[s, j, pl.ds(k * 16, 16)] = raw + slot * _VOCAB

    def fire_gathers(s):
        for j in range(_CHUNK):
            pltpu.async_copy(
                tab_hbm.at[idx_v.at[s, j]], rows_v.at[s, j], gsem.at[s]
            )

    def drain_gathers(s):
        for j in range(_CHUNK):
            pltpu.make_async_copy(
                tab_hbm.at[idx_v.at[s, j]], rows_v.at[s, j], gsem.at[s]
            ).wait()

    def fire_write(g, s):
        row0 = wstart + g * _CHUNK
        pltpu.async_copy(rows_v.at[s], out_hbm.at[pl.ds(row0, _CHUNK)], wsem.at[s])

    def wait_write(g, s):
        row0 = wstart + g * _CHUNK
        pltpu.make_async_copy(
            rows_v.at[s], out_hbm.at[pl.ds(row0, _CHUNK)], wsem.at[s]
        ).wait()

    def step(g):
        s = lax.rem(g, 2)
        p = 1 - s
        @pl.when(g >= 2)
        def _():
            wait_write(g - 2, s)

        stage_indices(g, s)
        fire_gathers(s)

        @pl.when(g >= 1)
        def _():
            drain_gathers(p)
            fire_write(g - 1, p)

    pl.loop(0, _NCHUNK)(step)

    s_last = (_NCHUNK - 1) % 2
    drain_gathers(s_last)
    fire_write(_NCHUNK - 1, s_last)
    wait_write(_NCHUNK - 2, 1 - s_last)
    wait_write(_NCHUNK - 1, s_last)


def kernel(input, tables):
    tab = tables.reshape(_N_SLOTS * _VOCAB, _EMB)
    idx = input.reshape(_NROW128, _IW)
    mesh = plsc.VectorSubcoreMesh(core_axis_name="c", subcore_axis_name="s")
    k = functools.partial(
        pl.kernel,
        out_type=jax.ShapeDtypeStruct((_NROW128, _IW, _EMB), jnp.float32),
        mesh=mesh,
        scratch_types=[
            pltpu.VMEM((2, _CHUNK, _IW), jnp.int32),
            pltpu.VMEM((2, _CHUNK, _IW, _EMB), jnp.float32),
            pltpu.SemaphoreType.DMA((2,)),
            pltpu.SemaphoreType.DMA((2,)),
        ],
        compiler_params=pltpu.CompilerParams(use_tc_tiling_on_sc=False),
    )(_body)
    out = k(tab, idx)
    return out.reshape(_BATCH, _N_SLOTS, _EMB)


# speedup vs baseline: 1.0058x; 1.0000x over previous
import functools

import jax
import jax.numpy as jnp
from jax import lax
from jax.experimental import pallas as pl
from jax.experimental.pallas import tpu as pltpu
from jax.experimental.pallas import tpu_sc as plsc

_BATCH = 16384
_N_SLOTS = 26
_VOCAB = 100000
_EMB = 32

_NW = 32
_BPW = _BATCH // _NW
_IW = 128
_NG = _BPW // _IW


def _body(tab_hbm, idxt_hbm, out_hbm, col_v, rows_v, gsem, wsem):
    nc = 2
    wid = lax.axis_index("s") * nc + lax.axis_index("c")
    b0 = wid * _BPW

    def stage(i, s):
        pltpu.sync_copy(idxt_hbm.at[i, pl.ds(b0, _BPW)], col_v.at[s])

    def fire_gathers(i, s):
        for j in range(_NG):
            pltpu.async_copy(
                tab_hbm.at[i].at[col_v.at[s, pl.ds(j * _IW, _IW)]],
                rows_v.at[s, pl.ds(j * _IW, _IW)],
                gsem.at[s],
            )

    def drain_gathers(i, s):
        for j in range(_NG):
            pltpu.make_async_copy(
                tab_hbm.at[i].at[col_v.at[s, pl.ds(j * _IW, _IW)]],
                rows_v.at[s, pl.ds(j * _IW, _IW)],
                gsem.at[s],
            ).wait()

    def fire_write(i, s):
        pltpu.async_copy(
            rows_v.at[s], out_hbm.at[pl.ds(b0, _BPW), i], wsem.at[s]
        )

    def wait_write(i, s):
        pltpu.make_async_copy(
            rows_v.at[s], out_hbm.at[pl.ds(b0, _BPW), i], wsem.at[s]
        ).wait()

    def step(i):
        s = lax.rem(i, 2)
        p = 1 - s
        @pl.when(i >= 2)
        def _():
            wait_write(i - 2, s)

        stage(i, s)
        fire_gathers(i, s)

        @pl.when(i >= 1)
        def _():
            drain_gathers(i - 1, p)
            fire_write(i - 1, p)

    pl.loop(0, _N_SLOTS)(step)

    s_last = (_N_SLOTS - 1) % 2
    drain_gathers(_N_SLOTS - 1, s_last)
    fire_write(_N_SLOTS - 1, s_last)
    wait_write(_N_SLOTS - 2, 1 - s_last)
    wait_write(_N_SLOTS - 1, s_last)


def kernel(input, tables):
    idxt = input.T
    mesh = plsc.VectorSubcoreMesh(core_axis_name="c", subcore_axis_name="s")
    k = functools.partial(
        pl.kernel,
        out_type=jax.ShapeDtypeStruct((_BATCH, _N_SLOTS, _EMB), jnp.float32),
        mesh=mesh,
        scratch_types=[
            pltpu.VMEM((2, _BPW), jnp.int32),
            pltpu.VMEM((2, _BPW, _EMB), jnp.float32),
            pltpu.SemaphoreType.DMA((2,)),
            pltpu.SemaphoreType.DMA((2,)),
        ],
        compiler_params=pltpu.CompilerParams(use_tc_tiling_on_sc=False),
    )(_body)
    return k(tables, idxt)
